# Initial kernel scaffold; baseline (speedup 1.0000x reference)
import jax
import jax.numpy as jnp
from jax.experimental import pallas as pl


def kernel(memory, key, beta, Wk, bk, Wh, bh, Wo, bo):
    raise NotImplementedError("write your pallas kernel here")


# speedup vs baseline: 3.3094x; 3.3094x over previous
import functools

import jax
import jax.numpy as jnp
from jax import lax
from jax.experimental import pallas as pl
from jax.experimental.pallas import tpu as pltpu

_B, _M, _I, _H = 16, 32768, 512, 8
_HD = _I // _H
_EPS = 1e-8
_MBLK = 2048
_NM = _M // _MBLK
_WCHUNK = 8192
_NW = _M // _WCHUNK
_NEG = -1e30


def _head_mask():
    lane = lax.broadcasted_iota(jnp.int32, (_H, _I), 1)
    head = lax.broadcasted_iota(jnp.int32, (_H, _I), 0)
    return jnp.where(lane // _HD == head, 1.0, 0.0).astype(jnp.float32)


def _proj_kernel(key_ref, wk_ref, bk_ref, wh_ref, bh_ref, khat_ref):
    pk = jnp.tanh(
        lax.dot_general(key_ref[...], wk_ref[...], (((1,), (1,)), ((), ())),
                        precision=lax.Precision.HIGHEST)
        + bk_ref[...])
    parts = []
    for h in range(_H):
        ph = pk[:, h * _HD:(h + 1) * _HD]
        y = jnp.tanh(
            lax.dot_general(ph, wh_ref[h], (((1,), (1,)), ((), ())),
                            precision=lax.Precision.HIGHEST)
            + bh_ref[h:h + 1, :])
        nrm = jnp.sqrt(jnp.sum(y * y, axis=-1, keepdims=True))
        parts.append(y / (nrm + _EPS))
    kn = jnp.concatenate(parts, axis=-1)
    khat_ref[...] = kn[:, None, :] * _head_mask()[None, :, :]


def _main_kernel(mem_ref, khat_ref, st_ref, beta_ref,
                 logits_ref, acc_ref, mrow_ref, srow_ref):
    j = pl.program_id(1)

    @pl.when(j == 0)
    def _():
        acc_ref[0] = jnp.zeros_like(acc_ref[0])
        mrow_ref[0] = jnp.full_like(mrow_ref[0], _NEG)
        srow_ref[0] = jnp.zeros_like(srow_ref[0])

    mem = mem_ref[0]
    kh = khat_ref[0]
    dots = lax.dot_general(kh, mem, (((1,), (1,)), ((), ())),
                           preferred_element_type=jnp.float32)
    msq = mem * mem
    nsq = lax.dot_general(st_ref[...], msq, (((1,), (1,)), ((), ())),
                          preferred_element_type=jnp.float32)
    sim = dots / (jnp.sqrt(nsq) + _EPS)
    logits = beta_ref[0] * sim
    logits_ref[0] = logits

    m_prev = mrow_ref[0][:, 0:1]
    s_prev = srow_ref[0][:, 0:1]
    bmax = jnp.max(logits, axis=-1, keepdims=True)
    m_new = jnp.maximum(m_prev, bmax)
    corr = jnp.exp(m_prev - m_new)
    p = jnp.exp(logits - m_new)
    s_new = s_prev * corr + jnp.sum(p, axis=-1, keepdims=True)
    pacc = lax.dot_general(p, mem, (((1,), (0,)), ((), ())),
                           preferred_element_type=jnp.float32)
    acc_ref[0] = acc_ref[0] * corr + pacc
    mrow_ref[0] = jnp.broadcast_to(m_new, (_H, 128))
    srow_ref[0] = jnp.broadcast_to(s_new, (_H, 128))


def _wnorm_kernel(logits_ref, mrow_ref, srow_ref, w_ref):
    m = mrow_ref[0][:, 0:1]
    s = srow_ref[0][:, 0:1]
    w_ref[0] = jnp.exp(logits_ref[0] - m) / s


def _out_kernel(acc_ref, srow_ref, st_ref, wo_ref, bo_ref, out_ref):
    s = srow_ref[...][:, :, 0:1]
    reads = jnp.sum(acc_ref[...] * st_ref[...][None, :, :] / s, axis=1)
    out_ref[...] = lax.dot_general(
        reads, wo_ref[...], (((1,), (1,)), ((), ())),
        precision=lax.Precision.HIGHEST) + bo_ref[...]


@functools.partial(jax.jit, static_argnames=("interpret",))
def kernel(memory, key, beta, Wk, bk, Wh, bh, Wo, bo, interpret=False):
    f32 = jnp.float32
    khat = pl.pallas_call(
        _proj_kernel,
        out_shape=jax.ShapeDtypeStruct((_B, _H, _I), f32),
        interpret=interpret,
        name="mhca_proj",
    )(key, Wk, bk.reshape(1, _I), Wh, bh)

    st = _head_mask()
    betar = beta.reshape(_B, _H, 1)

    grid = (_B, _NM)
    logits, acc, mrow, srow = pl.pallas_call(
        _main_kernel,
        grid=grid,
        in_specs=[
            pl.BlockSpec((1, _MBLK, _I), lambda b, j: (b, j, 0)),
            pl.BlockSpec((1, _H, _I), lambda b, j: (b, 0, 0)),
            pl.BlockSpec((_H, _I), lambda b, j: (0, 0)),
            pl.BlockSpec((1, _H, 1), lambda b, j: (b, 0, 0)),
        ],
        out_specs=[
            pl.BlockSpec((1, _H, _MBLK), lambda b, j: (b, 0, j)),
            pl.BlockSpec((1, _H, _I), lambda b, j: (b, 0, 0)),
            pl.BlockSpec((1, _H, 128), lambda b, j: (b, 0, 0)),
            pl.BlockSpec((1, _H, 128), lambda b, j: (b, 0, 0)),
        ],
        out_shape=[
            jax.ShapeDtypeStruct((_B, _H, _M), f32),
            jax.ShapeDtypeStruct((_B, _H, _I), f32),
            jax.ShapeDtypeStruct((_B, _H, 128), f32),
            jax.ShapeDtypeStruct((_B, _H, 128), f32),
        ],
        compiler_params=pltpu.CompilerParams(
            dimension_semantics=("parallel", "arbitrary"),
        ),
        interpret=interpret,
        name="mhca_main",
    )(memory, khat, st, betar)

    w = pl.pallas_call(
        _wnorm_kernel,
        grid=(_B, _NW),
        in_specs=[
            pl.BlockSpec((1, _H, _WCHUNK), lambda b, j: (b, 0, j)),
            pl.BlockSpec((1, _H, 128), lambda b, j: (b, 0, 0)),
            pl.BlockSpec((1, _H, 128), lambda b, j: (b, 0, 0)),
        ],
        out_specs=pl.BlockSpec((1, _H, _WCHUNK), lambda b, j: (b, 0, j)),
        out_shape=jax.ShapeDtypeStruct((_B, _H, _M), f32),
        compiler_params=pltpu.CompilerParams(
            dimension_semantics=("parallel", "arbitrary"),
        ),
        interpret=interpret,
        name="mhca_wnorm",
    )(logits, mrow, srow)

    out = pl.pallas_call(
        _out_kernel,
        out_shape=jax.ShapeDtypeStruct((_B, _I), f32),
        interpret=interpret,
        name="mhca_out",
    )(acc, srow, st, Wo, bo.reshape(1, _I))

    return out, w


# speedup vs baseline: 3.4816x; 1.0520x over previous
import functools

import jax
import jax.numpy as jnp
from jax import lax
from jax.experimental import pallas as pl
from jax.experimental.pallas import tpu as pltpu

_B, _M, _I, _H = 16, 32768, 512, 8
_HD = _I // _H
_EPS = 1e-8
_MBLK = 2048
_NM = _M // _MBLK
_WCHUNK = 8192
_NW = _M // _WCHUNK


def _head_mask():
    lane = lax.broadcasted_iota(jnp.int32, (_H, _I), 1)
    head = lax.broadcasted_iota(jnp.int32, (_H, _I), 0)
    return jnp.where(lane // _HD == head, 1.0, 0.0).astype(jnp.float32)


def _proj_kernel(key_ref, wk_ref, bk_ref, wh_ref, bh_ref, khat_ref):
    pk = jnp.tanh(
        lax.dot_general(key_ref[...], wk_ref[...], (((1,), (1,)), ((), ())),
                        precision=lax.Precision.HIGHEST)
        + bk_ref[...])
    parts = []
    for h in range(_H):
        ph = pk[:, h * _HD:(h + 1) * _HD]
        y = jnp.tanh(
            lax.dot_general(ph, wh_ref[h], (((1,), (1,)), ((), ())),
                            precision=lax.Precision.HIGHEST)
            + bh_ref[h:h + 1, :])
        nrm = jnp.sqrt(jnp.sum(y * y, axis=-1, keepdims=True))
        parts.append(y / (nrm + _EPS))
    kn = jnp.concatenate(parts, axis=-1)
    khat_ref[...] = kn[:, None, :] * _head_mask()[None, :, :]


def _main_kernel(mem_ref, khat_ref, st_ref, beta_ref,
                 logits_ref, acc_ref, srow_ref):
    j = pl.program_id(1)

    @pl.when(j == 0)
    def _():
        acc_ref[0] = jnp.zeros_like(acc_ref[0])
        srow_ref[0] = jnp.zeros_like(srow_ref[0])

    mem_bf = mem_ref[0].astype(jnp.bfloat16)
    msq_bf = mem_bf * mem_bf
    kh = khat_ref[0]
    dots = lax.dot_general(kh, mem_bf, (((1,), (1,)), ((), ())),
                           preferred_element_type=jnp.float32)
    nsq = lax.dot_general(st_ref[...], msq_bf, (((1,), (1,)), ((), ())),
                          preferred_element_type=jnp.float32)
    sim = dots / (jnp.sqrt(nsq) + _EPS)
    logits = beta_ref[0] * sim
    logits_ref[0] = logits

    p = jnp.exp(logits)
    srow_ref[0] += jnp.broadcast_to(
        jnp.sum(p, axis=-1, keepdims=True), (_H, 128))
    pacc = lax.dot_general(p.astype(jnp.bfloat16), mem_bf,
                           (((1,), (0,)), ((), ())),
                           preferred_element_type=jnp.float32)
    acc_ref[0] += pacc


def _wnorm_kernel(logits_ref, srow_ref, w_ref):
    s = srow_ref[0][:, 0:1]
    w_ref[0] = jnp.exp(logits_ref[0]) / s


def _out_kernel(acc_ref, srow_ref, st_ref, wo_ref, bo_ref, out_ref):
    s = srow_ref[...][:, :, 0:1]
    reads = jnp.sum(acc_ref[...] * st_ref[...][None, :, :] / s, axis=1)
    out_ref[...] = lax.dot_general(
        reads, wo_ref[...], (((1,), (1,)), ((), ())),
        precision=lax.Precision.HIGHEST) + bo_ref[...]


@functools.partial(jax.jit, static_argnames=("interpret",))
def kernel(memory, key, beta, Wk, bk, Wh, bh, Wo, bo, interpret=False):
    f32 = jnp.float32
    khat = pl.pallas_call(
        _proj_kernel,
        out_shape=jax.ShapeDtypeStruct((_B, _H, _I), f32),
        interpret=interpret,
        name="mhca_proj",
    )(key, Wk, bk.reshape(1, _I), Wh, bh)

    st = _head_mask()
    betar = beta.reshape(_B, _H, 1)

    grid = (_B, _NM)
    logits, acc, srow = pl.pallas_call(
        _main_kernel,
        grid=grid,
        in_specs=[
            pl.BlockSpec((1, _MBLK, _I), lambda b, j: (b, j, 0)),
            pl.BlockSpec((1, _H, _I), lambda b, j: (b, 0, 0)),
            pl.BlockSpec((_H, _I), lambda b, j: (0, 0)),
            pl.BlockSpec((1, _H, 1), lambda b, j: (b, 0, 0)),
        ],
        out_specs=[
            pl.BlockSpec((1, _H, _MBLK), lambda b, j: (b, 0, j)),
            pl.BlockSpec((1, _H, _I), lambda b, j: (b, 0, 0)),
            pl.BlockSpec((1, _H, 128), lambda b, j: (b, 0, 0)),
        ],
        out_shape=[
            jax.ShapeDtypeStruct((_B, _H, _M), f32),
            jax.ShapeDtypeStruct((_B, _H, _I), f32),
            jax.ShapeDtypeStruct((_B, _H, 128), f32),
        ],
        compiler_params=pltpu.CompilerParams(
            dimension_semantics=("parallel", "arbitrary"),
        ),
        interpret=interpret,
        name="mhca_main",
    )(memory, khat.astype(jnp.bfloat16), st.astype(jnp.bfloat16), betar)

    w = pl.pallas_call(
        _wnorm_kernel,
        grid=(_B, _NW),
        in_specs=[
            pl.BlockSpec((1, _H, _WCHUNK), lambda b, j: (b, 0, j)),
            pl.BlockSpec((1, _H, 128), lambda b, j: (b, 0, 0)),
        ],
        out_specs=pl.BlockSpec((1, _H, _WCHUNK), lambda b, j: (b, 0, j)),
        out_shape=jax.ShapeDtypeStruct((_B, _H, _M), f32),
        compiler_params=pltpu.CompilerParams(
            dimension_semantics=("parallel", "arbitrary"),
        ),
        interpret=interpret,
        name="mhca_wnorm",
    )(logits, srow)

    out = pl.pallas_call(
        _out_kernel,
        out_shape=jax.ShapeDtypeStruct((_B, _I), f32),
        interpret=interpret,
        name="mhca_out",
    )(acc, srow, st, Wo, bo.reshape(1, _I))

    return out, w


# speedup vs baseline: 3.9951x; 1.1475x over previous
import functools

import jax
import jax.numpy as jnp
from jax import lax
from jax.experimental import pallas as pl
from jax.experimental.pallas import tpu as pltpu

_B, _M, _I, _H = 16, 32768, 512, 8
_HD = _I // _H
_EPS = 1e-8
_MBLK = 4096
_NM = _M // _MBLK
_WCHUNK = 8192
_NW = _M // _WCHUNK


def _head_mask():
    lane = lax.broadcasted_iota(jnp.int32, (_H, _I), 1)
    head = lax.broadcasted_iota(jnp.int32, (_H, _I), 0)
    return jnp.where(lane // _HD == head, 1.0, 0.0).astype(jnp.float32)


def _proj_kernel(key_ref, wk_ref, bk_ref, wh_ref, bh_ref, khat_ref):
    pk = jnp.tanh(
        lax.dot_general(key_ref[...], wk_ref[...], (((1,), (1,)), ((), ())),
                        precision=lax.Precision.HIGHEST)
        + bk_ref[...])
    parts = []
    for h in range(_H):
        ph = pk[:, h * _HD:(h + 1) * _HD]
        y = jnp.tanh(
            lax.dot_general(ph, wh_ref[h], (((1,), (1,)), ((), ())),
                            precision=lax.Precision.HIGHEST)
            + bh_ref[h:h + 1, :])
        nrm = jnp.sqrt(jnp.sum(y * y, axis=-1, keepdims=True))
        parts.append(y / (nrm + _EPS))
    kn = jnp.concatenate(parts, axis=-1)
    khat_ref[...] = kn[:, None, :] * _head_mask()[None, :, :]


def _main_kernel(mem_ref, khat_ref, st_ref, beta_ref,
                 logits_ref, acc_ref, srow_ref):
    j = pl.program_id(1)

    @pl.when(j == 0)
    def _():
        acc_ref[0] = jnp.zeros_like(acc_ref[0])
        srow_ref[0] = jnp.zeros_like(srow_ref[0])

    mem_bf = mem_ref[0].astype(jnp.bfloat16)
    msq_bf = mem_bf * mem_bf
    kh = khat_ref[0]
    dots = lax.dot_general(kh, mem_bf, (((1,), (1,)), ((), ())),
                           preferred_element_type=jnp.float32)
    nsq = lax.dot_general(st_ref[...], msq_bf, (((1,), (1,)), ((), ())),
                          preferred_element_type=jnp.float32)
    sim = dots / (jnp.sqrt(nsq) + _EPS)
    logits = beta_ref[0] * sim
    logits_ref[0] = logits

    p = jnp.exp(logits)
    srow_ref[0] += jnp.broadcast_to(
        jnp.sum(p, axis=-1, keepdims=True), (_H, 128))
    pacc = lax.dot_general(p.astype(jnp.bfloat16), mem_bf,
                           (((1,), (0,)), ((), ())),
                           preferred_element_type=jnp.float32)
    acc_ref[0] += pacc


def _wnorm_kernel(logits_ref, srow_ref, w_ref):
    s = srow_ref[0][:, 0:1]
    w_ref[0] = jnp.exp(logits_ref[0]) / s


def _out_kernel(acc_ref, srow_ref, st_ref, wo_ref, bo_ref, out_ref):
    s = srow_ref[...][:, :, 0:1]
    reads = jnp.sum(acc_ref[...] * st_ref[...][None, :, :] / s, axis=1)
    out_ref[...] = lax.dot_general(
        reads, wo_ref[...], (((1,), (1,)), ((), ())),
        precision=lax.Precision.HIGHEST) + bo_ref[...]


@functools.partial(jax.jit, static_argnames=("interpret",))
def kernel(memory, key, beta, Wk, bk, Wh, bh, Wo, bo, interpret=False):
    f32 = jnp.float32
    khat = pl.pallas_call(
        _proj_kernel,
        out_shape=jax.ShapeDtypeStruct((_B, _H, _I), f32),
        interpret=interpret,
        name="mhca_proj",
    )(key, Wk, bk.reshape(1, _I), Wh, bh)

    st = _head_mask()
    betar = beta.reshape(_B, _H, 1)

    grid = (_B, _NM)
    logits, acc, srow = pl.pallas_call(
        _main_kernel,
        grid=grid,
        in_specs=[
            pl.BlockSpec((1, _MBLK, _I), lambda b, j: (b, j, 0)),
            pl.BlockSpec((1, _H, _I), lambda b, j: (b, 0, 0)),
            pl.BlockSpec((_H, _I), lambda b, j: (0, 0)),
            pl.BlockSpec((1, _H, 1), lambda b, j: (b, 0, 0)),
        ],
        out_specs=[
            pl.BlockSpec((1, _H, _MBLK), lambda b, j: (b, 0, j)),
            pl.BlockSpec((1, _H, _I), lambda b, j: (b, 0, 0)),
            pl.BlockSpec((1, _H, 128), lambda b, j: (b, 0, 0)),
        ],
        out_shape=[
            jax.ShapeDtypeStruct((_B, _H, _M), f32),
            jax.ShapeDtypeStruct((_B, _H, _I), f32),
            jax.ShapeDtypeStruct((_B, _H, 128), f32),
        ],
        compiler_params=pltpu.CompilerParams(
            dimension_semantics=("parallel", "arbitrary"),
            vmem_limit_bytes=50 * 1024 * 1024,
        ),
        interpret=interpret,
        name="mhca_main",
    )(memory, khat.astype(jnp.bfloat16), st.astype(jnp.bfloat16), betar)

    w = pl.pallas_call(
        _wnorm_kernel,
        grid=(_B, _NW),
        in_specs=[
            pl.BlockSpec((1, _H, _WCHUNK), lambda b, j: (b, 0, j)),
            pl.BlockSpec((1, _H, 128), lambda b, j: (b, 0, 0)),
        ],
        out_specs=pl.BlockSpec((1, _H, _WCHUNK), lambda b, j: (b, 0, j)),
        out_shape=jax.ShapeDtypeStruct((_B, _H, _M), f32),
        compiler_params=pltpu.CompilerParams(
            dimension_semantics=("parallel", "arbitrary"),
        ),
        interpret=interpret,
        name="mhca_wnorm",
    )(logits, srow)

    out = pl.pallas_call(
        _out_kernel,
        out_shape=jax.ShapeDtypeStruct((_B, _I), f32),
        interpret=interpret,
        name="mhca_out",
    )(acc, srow, st, Wo, bo.reshape(1, _I))

    return out, w


# speedup vs baseline: 4.3533x; 1.0897x over previous
import functools

import jax
import jax.numpy as jnp
from jax import lax
from jax.experimental import pallas as pl
from jax.experimental.pallas import tpu as pltpu

_B, _M, _I, _H = 16, 32768, 512, 8
_HD = _I // _H
_EPS = 1e-8
_MBLK = 8192
_NM = _M // _MBLK
_WCHUNK = 8192
_NW = _M // _WCHUNK


def _head_mask():
    lane = lax.broadcasted_iota(jnp.int32, (_H, _I), 1)
    head = lax.broadcasted_iota(jnp.int32, (_H, _I), 0)
    return jnp.where(lane // _HD == head, 1.0, 0.0).astype(jnp.float32)


def _proj_kernel(key_ref, wk_ref, bk_ref, wh_ref, bh_ref, khat_ref):
    pk = jnp.tanh(
        lax.dot_general(key_ref[...], wk_ref[...], (((1,), (1,)), ((), ())),
                        precision=lax.Precision.HIGHEST)
        + bk_ref[...])
    parts = []
    for h in range(_H):
        ph = pk[:, h * _HD:(h + 1) * _HD]
        y = jnp.tanh(
            lax.dot_general(ph, wh_ref[h], (((1,), (1,)), ((), ())),
                            precision=lax.Precision.HIGHEST)
            + bh_ref[h:h + 1, :])
        nrm = jnp.sqrt(jnp.sum(y * y, axis=-1, keepdims=True))
        parts.append(y / (nrm + _EPS))
    kn = jnp.concatenate(parts, axis=-1)
    khat_ref[...] = kn[:, None, :] * _head_mask()[None, :, :]


def _main_kernel(mem_ref, khat_ref, st_ref, beta_ref,
                 logits_ref, acc_ref, srow_ref):
    j = pl.program_id(1)

    @pl.when(j == 0)
    def _():
        acc_ref[0] = jnp.zeros_like(acc_ref[0])
        srow_ref[0] = jnp.zeros_like(srow_ref[0])

    mem_bf = mem_ref[0].astype(jnp.bfloat16)
    msq_bf = mem_bf * mem_bf
    kh = khat_ref[0]
    dots = lax.dot_general(kh, mem_bf, (((1,), (1,)), ((), ())),
                           preferred_element_type=jnp.float32)
    nsq = lax.dot_general(st_ref[...], msq_bf, (((1,), (1,)), ((), ())),
                          preferred_element_type=jnp.float32)
    sim = dots / (jnp.sqrt(nsq) + _EPS)
    logits = beta_ref[0] * sim
    logits_ref[0] = logits

    p = jnp.exp(logits)
    srow_ref[0] += jnp.broadcast_to(
        jnp.sum(p, axis=-1, keepdims=True), (_H, 128))
    pacc = lax.dot_general(p.astype(jnp.bfloat16), mem_bf,
                           (((1,), (0,)), ((), ())),
                           preferred_element_type=jnp.float32)
    acc_ref[0] += pacc


def _wnorm_kernel(logits_ref, srow_ref, w_ref):
    s = srow_ref[0][:, 0:1]
    w_ref[0] = jnp.exp(logits_ref[0]) / s


def _out_kernel(acc_ref, srow_ref, st_ref, wo_ref, bo_ref, out_ref):
    s = srow_ref[...][:, :, 0:1]
    reads = jnp.sum(acc_ref[...] * st_ref[...][None, :, :] / s, axis=1)
    out_ref[...] = lax.dot_general(
        reads, wo_ref[...], (((1,), (1,)), ((), ())),
        precision=lax.Precision.HIGHEST) + bo_ref[...]


@functools.partial(jax.jit, static_argnames=("interpret",))
def kernel(memory, key, beta, Wk, bk, Wh, bh, Wo, bo, interpret=False):
    f32 = jnp.float32
    khat = pl.pallas_call(
        _proj_kernel,
        out_shape=jax.ShapeDtypeStruct((_B, _H, _I), f32),
        interpret=interpret,
        name="mhca_proj",
    )(key, Wk, bk.reshape(1, _I), Wh, bh)

    st = _head_mask()
    betar = beta.reshape(_B, _H, 1)

    grid = (_B, _NM)
    logits, acc, srow = pl.pallas_call(
        _main_kernel,
        grid=grid,
        in_specs=[
            pl.BlockSpec((1, _MBLK, _I), lambda b, j: (b, j, 0)),
            pl.BlockSpec((1, _H, _I), lambda b, j: (b, 0, 0)),
            pl.BlockSpec((_H, _I), lambda b, j: (0, 0)),
            pl.BlockSpec((1, _H, 1), lambda b, j: (b, 0, 0)),
        ],
        out_specs=[
            pl.BlockSpec((1, _H, _MBLK), lambda b, j: (b, 0, j)),
            pl.BlockSpec((1, _H, _I), lambda b, j: (b, 0, 0)),
            pl.BlockSpec((1, _H, 128), lambda b, j: (b, 0, 0)),
        ],
        out_shape=[
            jax.ShapeDtypeStruct((_B, _H, _M), f32),
            jax.ShapeDtypeStruct((_B, _H, _I), f32),
            jax.ShapeDtypeStruct((_B, _H, 128), f32),
        ],
        compiler_params=pltpu.CompilerParams(
            dimension_semantics=("parallel", "arbitrary"),
            vmem_limit_bytes=50 * 1024 * 1024,
        ),
        interpret=interpret,
        name="mhca_main",
    )(memory, khat.astype(jnp.bfloat16), st.astype(jnp.bfloat16), betar)

    w = pl.pallas_call(
        _wnorm_kernel,
        grid=(_B, _NW),
        in_specs=[
            pl.BlockSpec((1, _H, _WCHUNK), lambda b, j: (b, 0, j)),
            pl.BlockSpec((1, _H, 128), lambda b, j: (b, 0, 0)),
        ],
        out_specs=pl.BlockSpec((1, _H, _WCHUNK), lambda b, j: (b, 0, j)),
        out_shape=jax.ShapeDtypeStruct((_B, _H, _M), f32),
        compiler_params=pltpu.CompilerParams(
            dimension_semantics=("parallel", "arbitrary"),
        ),
        interpret=interpret,
        name="mhca_wnorm",
    )(logits, srow)

    out = pl.pallas_call(
        _out_kernel,
        out_shape=jax.ShapeDtypeStruct((_B, _I), f32),
        interpret=interpret,
        name="mhca_out",
    )(acc, srow, st, Wo, bo.reshape(1, _I))

    return out, w


# speedup vs baseline: 4.5623x; 1.0480x over previous
import functools

import jax
import jax.numpy as jnp
from jax import lax
from jax.experimental import pallas as pl
from jax.experimental.pallas import tpu as pltpu

_B, _M, _I, _H = 16, 32768, 512, 8
_HD = _I // _H
_EPS = 1e-8
_MBLK = 8192
_NM = _M // _MBLK
_WCHUNK = 16384
_NW = _M // _WCHUNK


def _head_mask():
    lane = lax.broadcasted_iota(jnp.int32, (_H, _I), 1)
    head = lax.broadcasted_iota(jnp.int32, (_H, _I), 0)
    return jnp.where(lane // _HD == head, 1.0, 0.0).astype(jnp.float32)


def _proj_kernel(key_ref, wk_ref, bk_ref, wh_ref, bh_ref, khat_ref):
    pk = jnp.tanh(
        lax.dot_general(key_ref[...], wk_ref[...], (((1,), (1,)), ((), ())),
                        precision=lax.Precision.HIGHEST)
        + bk_ref[...])
    parts = []
    for h in range(_H):
        ph = pk[:, h * _HD:(h + 1) * _HD]
        y = jnp.tanh(
            lax.dot_general(ph, wh_ref[h], (((1,), (1,)), ((), ())),
                            precision=lax.Precision.HIGHEST)
            + bh_ref[h:h + 1, :])
        nrm = jnp.sqrt(jnp.sum(y * y, axis=-1, keepdims=True))
        parts.append(y / (nrm + _EPS))
    kn = jnp.concatenate(parts, axis=-1)
    khat_ref[...] = kn[:, None, :] * _head_mask()[None, :, :]


def _main_kernel(mem_ref, khat_ref, st_ref, beta_ref,
                 logits_ref, acc_ref, srow_ref):
    j = pl.program_id(1)

    @pl.when(j == 0)
    def _():
        acc_ref[0] = jnp.zeros_like(acc_ref[0])
        srow_ref[0] = jnp.zeros_like(srow_ref[0])

    mem_bf = mem_ref[0].astype(jnp.bfloat16)
    msq_bf = mem_bf * mem_bf
    kh = khat_ref[0]
    dots = lax.dot_general(kh, mem_bf, (((1,), (1,)), ((), ())),
                           preferred_element_type=jnp.float32)
    nsq = lax.dot_general(st_ref[...], msq_bf, (((1,), (1,)), ((), ())),
                          preferred_element_type=jnp.float32)
    sim = dots / (jnp.sqrt(nsq) + _EPS)
    logits = beta_ref[0] * sim
    logits_ref[0] = logits.astype(jnp.bfloat16)

    p = jnp.exp(logits)
    srow_ref[0] += jnp.broadcast_to(
        jnp.sum(p, axis=-1, keepdims=True), (_H, 128))
    pacc = lax.dot_general(p.astype(jnp.bfloat16), mem_bf,
                           (((1,), (0,)), ((), ())),
                           preferred_element_type=jnp.float32)
    acc_ref[0] += pacc


def _wnorm_kernel(logits_ref, srow_ref, w_ref):
    s = srow_ref[0][:, 0:1]
    w_ref[0] = jnp.exp(logits_ref[0].astype(jnp.float32)) / s


def _out_kernel(acc_ref, srow_ref, st_ref, wo_ref, bo_ref, out_ref):
    s = srow_ref[...][:, :, 0:1]
    reads = jnp.sum(acc_ref[...] * st_ref[...][None, :, :] / s, axis=1)
    out_ref[...] = lax.dot_general(
        reads, wo_ref[...], (((1,), (1,)), ((), ())),
        precision=lax.Precision.HIGHEST) + bo_ref[...]


@functools.partial(jax.jit, static_argnames=("interpret",))
def kernel(memory, key, beta, Wk, bk, Wh, bh, Wo, bo, interpret=False):
    f32 = jnp.float32
    khat = pl.pallas_call(
        _proj_kernel,
        out_shape=jax.ShapeDtypeStruct((_B, _H, _I), f32),
        interpret=interpret,
        name="mhca_proj",
    )(key, Wk, bk.reshape(1, _I), Wh, bh)

    st = _head_mask()
    betar = beta.reshape(_B, _H, 1)

    grid = (_B, _NM)
    logits, acc, srow = pl.pallas_call(
        _main_kernel,
        grid=grid,
        in_specs=[
            pl.BlockSpec((1, _MBLK, _I), lambda b, j: (b, j, 0)),
            pl.BlockSpec((1, _H, _I), lambda b, j: (b, 0, 0)),
            pl.BlockSpec((_H, _I), lambda b, j: (0, 0)),
            pl.BlockSpec((1, _H, 1), lambda b, j: (b, 0, 0)),
        ],
        out_specs=[
            pl.BlockSpec((1, _H, _MBLK), lambda b, j: (b, 0, j)),
            pl.BlockSpec((1, _H, _I), lambda b, j: (b, 0, 0)),
            pl.BlockSpec((1, _H, 128), lambda b, j: (b, 0, 0)),
        ],
        out_shape=[
            jax.ShapeDtypeStruct((_B, _H, _M), jnp.bfloat16),
            jax.ShapeDtypeStruct((_B, _H, _I), f32),
            jax.ShapeDtypeStruct((_B, _H, 128), f32),
        ],
        compiler_params=pltpu.CompilerParams(
            dimension_semantics=("parallel", "arbitrary"),
            vmem_limit_bytes=50 * 1024 * 1024,
        ),
        interpret=interpret,
        name="mhca_main",
    )(memory, khat.astype(jnp.bfloat16), st.astype(jnp.bfloat16), betar)

    w = pl.pallas_call(
        _wnorm_kernel,
        grid=(_B, _NW),
        in_specs=[
            pl.BlockSpec((1, _H, _WCHUNK), lambda b, j: (b, 0, j)),
            pl.BlockSpec((1, _H, 128), lambda b, j: (b, 0, 0)),
        ],
        out_specs=pl.BlockSpec((1, _H, _WCHUNK), lambda b, j: (b, 0, j)),
        out_shape=jax.ShapeDtypeStruct((_B, _H, _M), f32),
        compiler_params=pltpu.CompilerParams(
            dimension_semantics=("parallel", "arbitrary"),
        ),
        interpret=interpret,
        name="mhca_wnorm",
    )(logits, srow)

    out = pl.pallas_call(
        _out_kernel,
        out_shape=jax.ShapeDtypeStruct((_B, _I), f32),
        interpret=interpret,
        name="mhca_out",
    )(acc, srow, st, Wo, bo.reshape(1, _I))

    return out, w


# speedup vs baseline: 4.6405x; 1.0171x over previous
import functools

import jax
import jax.numpy as jnp
from jax import lax
from jax.experimental import pallas as pl
from jax.experimental.pallas import tpu as pltpu

_B, _M, _I, _H = 16, 32768, 512, 8
_HD = _I // _H
_EPS = 1e-8
_MBLK = 8192
_NM = _M // _MBLK
_WCHUNK = 32768
_NW = _M // _WCHUNK


def _head_mask():
    lane = lax.broadcasted_iota(jnp.int32, (_H, _I), 1)
    head = lax.broadcasted_iota(jnp.int32, (_H, _I), 0)
    return jnp.where(lane // _HD == head, 1.0, 0.0).astype(jnp.float32)


def _proj_kernel(key_ref, wk_ref, bk_ref, wh_ref, bh_ref, khat_ref):
    pk = jnp.tanh(
        lax.dot_general(key_ref[...], wk_ref[...], (((1,), (1,)), ((), ())),
                        precision=lax.Precision.HIGHEST)
        + bk_ref[...])
    parts = []
    for h in range(_H):
        ph = pk[:, h * _HD:(h + 1) * _HD]
        y = jnp.tanh(
            lax.dot_general(ph, wh_ref[h], (((1,), (1,)), ((), ())),
                            precision=lax.Precision.HIGHEST)
            + bh_ref[h:h + 1, :])
        nrm = jnp.sqrt(jnp.sum(y * y, axis=-1, keepdims=True))
        parts.append(y / (nrm + _EPS))
    kn = jnp.concatenate(parts, axis=-1)
    khat_ref[...] = kn[:, None, :] * _head_mask()[None, :, :]


def _main_kernel(mem_ref, khat_ref, st_ref, beta_ref,
                 logits_ref, acc_ref, srow_ref):
    j = pl.program_id(1)

    @pl.when(j == 0)
    def _():
        acc_ref[0] = jnp.zeros_like(acc_ref[0])
        srow_ref[0] = jnp.zeros_like(srow_ref[0])

    mem_bf = mem_ref[0].astype(jnp.bfloat16)
    msq_bf = mem_bf * mem_bf
    kh = khat_ref[0]
    dots = lax.dot_general(kh, mem_bf, (((1,), (1,)), ((), ())),
                           preferred_element_type=jnp.float32)
    nsq = lax.dot_general(st_ref[...], msq_bf, (((1,), (1,)), ((), ())),
                          preferred_element_type=jnp.float32)
    sim = dots / (jnp.sqrt(nsq) + _EPS)
    logits = beta_ref[0] * sim
    logits_ref[0] = logits.astype(jnp.bfloat16)

    p = jnp.exp(logits)
    srow_ref[0] += jnp.broadcast_to(
        jnp.sum(p, axis=-1, keepdims=True), (_H, 128))
    pacc = lax.dot_general(p.astype(jnp.bfloat16), mem_bf,
                           (((1,), (0,)), ((), ())),
                           preferred_element_type=jnp.float32)
    acc_ref[0] += pacc


def _wnorm_kernel(logits_ref, srow_ref, w_ref):
    s = srow_ref[0][:, 0:1]
    w_ref[0] = jnp.exp(logits_ref[0].astype(jnp.float32)) / s


def _out_kernel(acc_ref, srow_ref, st_ref, wo_ref, bo_ref, out_ref):
    s = srow_ref[...][:, :, 0:1]
    reads = jnp.sum(acc_ref[...] * st_ref[...][None, :, :] / s, axis=1)
    out_ref[...] = lax.dot_general(
        reads, wo_ref[...], (((1,), (1,)), ((), ())),
        precision=lax.Precision.HIGHEST) + bo_ref[...]


@functools.partial(jax.jit, static_argnames=("interpret",))
def kernel(memory, key, beta, Wk, bk, Wh, bh, Wo, bo, interpret=False):
    f32 = jnp.float32
    khat = pl.pallas_call(
        _proj_kernel,
        out_shape=jax.ShapeDtypeStruct((_B, _H, _I), f32),
        interpret=interpret,
        name="mhca_proj",
    )(key, Wk, bk.reshape(1, _I), Wh, bh)

    st = _head_mask()
    betar = beta.reshape(_B, _H, 1)

    grid = (_B, _NM)
    logits, acc, srow = pl.pallas_call(
        _main_kernel,
        grid=grid,
        in_specs=[
            pl.BlockSpec((1, _MBLK, _I), lambda b, j: (b, j, 0)),
            pl.BlockSpec((1, _H, _I), lambda b, j: (b, 0, 0)),
            pl.BlockSpec((_H, _I), lambda b, j: (0, 0)),
            pl.BlockSpec((1, _H, 1), lambda b, j: (b, 0, 0)),
        ],
        out_specs=[
            pl.BlockSpec((1, _H, _MBLK), lambda b, j: (b, 0, j)),
            pl.BlockSpec((1, _H, _I), lambda b, j: (b, 0, 0)),
            pl.BlockSpec((1, _H, 128), lambda b, j: (b, 0, 0)),
        ],
        out_shape=[
            jax.ShapeDtypeStruct((_B, _H, _M), jnp.bfloat16),
            jax.ShapeDtypeStruct((_B, _H, _I), f32),
            jax.ShapeDtypeStruct((_B, _H, 128), f32),
        ],
        compiler_params=pltpu.CompilerParams(
            dimension_semantics=("parallel", "arbitrary"),
            vmem_limit_bytes=50 * 1024 * 1024,
        ),
        interpret=interpret,
        name="mhca_main",
    )(memory, khat.astype(jnp.bfloat16), st.astype(jnp.bfloat16), betar)

    w = pl.pallas_call(
        _wnorm_kernel,
        grid=(_B, _NW),
        in_specs=[
            pl.BlockSpec((1, _H, _WCHUNK), lambda b, j: (b, 0, j)),
            pl.BlockSpec((1, _H, 128), lambda b, j: (b, 0, 0)),
        ],
        out_specs=pl.BlockSpec((1, _H, _WCHUNK), lambda b, j: (b, 0, j)),
        out_shape=jax.ShapeDtypeStruct((_B, _H, _M), f32),
        compiler_params=pltpu.CompilerParams(
            dimension_semantics=("parallel", "arbitrary"),
        ),
        interpret=interpret,
        name="mhca_wnorm",
    )(logits, srow)

    out = pl.pallas_call(
        _out_kernel,
        out_shape=jax.ShapeDtypeStruct((_B, _I), f32),
        interpret=interpret,
        name="mhca_out",
    )(acc, srow, st, Wo, bo.reshape(1, _I))

    return out, w


# speedup vs baseline: 4.6511x; 1.0023x over previous
import functools

import jax
import jax.numpy as jnp
import numpy as np
from jax import lax
from jax.experimental import pallas as pl
from jax.experimental.pallas import tpu as pltpu

_B, _M, _I, _H = 16, 32768, 512, 8
_HD = _I // _H
_EPS = 1e-8
_MBLK = 8192
_NM = _M // _MBLK
_WCHUNK = 32768
_NW = _M // _WCHUNK


def _head_mask():
    return (np.arange(_I)[None, :] // _HD == np.arange(_H)[:, None]).astype(np.float32)


def _proj_kernel(key_ref, wk_ref, bk_ref, wh_ref, bh_ref, khat_ref):
    pk = jnp.tanh(
        lax.dot_general(key_ref[...], wk_ref[...], (((1,), (1,)), ((), ())),
                        precision=lax.Precision.HIGHEST)
        + bk_ref[...])
    parts = []
    for h in range(_H):
        ph = pk[:, h * _HD:(h + 1) * _HD]
        y = jnp.tanh(
            lax.dot_general(ph, wh_ref[h], (((1,), (1,)), ((), ())),
                            precision=lax.Precision.HIGHEST)
            + bh_ref[h:h + 1, :])
        nrm = jnp.sqrt(jnp.sum(y * y, axis=-1, keepdims=True))
        parts.append(y / (nrm + _EPS))
    kn = jnp.concatenate(parts, axis=-1)
    lane = lax.broadcasted_iota(jnp.int32, (_H, _I), 1)
    head = lax.broadcasted_iota(jnp.int32, (_H, _I), 0)
    mask = jnp.where(lane // _HD == head, 1.0, 0.0).astype(jnp.float32)
    khat_ref[...] = (kn[:, None, :] * mask[None, :, :]).astype(jnp.bfloat16)


def _main_kernel(mem_ref, khat_ref, st_ref, beta_ref,
                 logits_ref, acc_ref, srow_ref):
    j = pl.program_id(1)

    @pl.when(j == 0)
    def _():
        acc_ref[0] = jnp.zeros_like(acc_ref[0])
        srow_ref[0] = jnp.zeros_like(srow_ref[0])

    mem_bf = mem_ref[0].astype(jnp.bfloat16)
    msq_bf = mem_bf * mem_bf
    kh = khat_ref[0]
    dots = lax.dot_general(kh, mem_bf, (((1,), (1,)), ((), ())),
                           preferred_element_type=jnp.float32)
    nsq = lax.dot_general(st_ref[...], msq_bf, (((1,), (1,)), ((), ())),
                          preferred_element_type=jnp.float32)
    sim = dots / (jnp.sqrt(nsq) + _EPS)
    logits = beta_ref[0] * sim
    logits_ref[0] = logits.astype(jnp.bfloat16)

    p = jnp.exp(logits)
    srow_ref[0] += jnp.broadcast_to(
        jnp.sum(p, axis=-1, keepdims=True), (_H, 128))
    pacc = lax.dot_general(p.astype(jnp.bfloat16), mem_bf,
                           (((1,), (0,)), ((), ())),
                           preferred_element_type=jnp.float32)
    acc_ref[0] += pacc


def _wnorm_kernel(logits_ref, srow_ref, w_ref):
    s = srow_ref[0][:, 0:1]
    w_ref[0] = jnp.exp(logits_ref[0].astype(jnp.float32)) / s


def _out_kernel(acc_ref, srow_ref, st_ref, wo_ref, bo_ref, out_ref):
    s = srow_ref[...][:, :, 0:1]
    reads = jnp.sum(acc_ref[...] * st_ref[...][None, :, :] / s, axis=1)
    out_ref[...] = lax.dot_general(
        reads, wo_ref[...], (((1,), (1,)), ((), ())),
        precision=lax.Precision.HIGHEST) + bo_ref[...]


@functools.partial(jax.jit, static_argnames=("interpret",))
def kernel(memory, key, beta, Wk, bk, Wh, bh, Wo, bo, interpret=False):
    f32 = jnp.float32
    khat = pl.pallas_call(
        _proj_kernel,
        out_shape=jax.ShapeDtypeStruct((_B, _H, _I), jnp.bfloat16),
        interpret=interpret,
        name="mhca_proj",
    )(key, Wk, bk.reshape(1, _I), Wh, bh)

    st = jnp.asarray(_head_mask())
    st_bf = jnp.asarray(_head_mask()).astype(jnp.bfloat16)
    betar = beta.reshape(_B, _H, 1)

    grid = (_B, _NM)
    logits, acc, srow = pl.pallas_call(
        _main_kernel,
        grid=grid,
        in_specs=[
            pl.BlockSpec((1, _MBLK, _I), lambda b, j: (b, j, 0)),
            pl.BlockSpec((1, _H, _I), lambda b, j: (b, 0, 0)),
            pl.BlockSpec((_H, _I), lambda b, j: (0, 0)),
            pl.BlockSpec((1, _H, 1), lambda b, j: (b, 0, 0)),
        ],
        out_specs=[
            pl.BlockSpec((1, _H, _MBLK), lambda b, j: (b, 0, j)),
            pl.BlockSpec((1, _H, _I), lambda b, j: (b, 0, 0)),
            pl.BlockSpec((1, _H, 128), lambda b, j: (b, 0, 0)),
        ],
        out_shape=[
            jax.ShapeDtypeStruct((_B, _H, _M), jnp.bfloat16),
            jax.ShapeDtypeStruct((_B, _H, _I), f32),
            jax.ShapeDtypeStruct((_B, _H, 128), f32),
        ],
        compiler_params=pltpu.CompilerParams(
            dimension_semantics=("parallel", "arbitrary"),
            vmem_limit_bytes=50 * 1024 * 1024,
        ),
        interpret=interpret,
        name="mhca_main",
    )(memory, khat, st_bf, betar)

    w = pl.pallas_call(
        _wnorm_kernel,
        grid=(_B, _NW),
        in_specs=[
            pl.BlockSpec((1, _H, _WCHUNK), lambda b, j: (b, 0, j)),
            pl.BlockSpec((1, _H, 128), lambda b, j: (b, 0, 0)),
        ],
        out_specs=pl.BlockSpec((1, _H, _WCHUNK), lambda b, j: (b, 0, j)),
        out_shape=jax.ShapeDtypeStruct((_B, _H, _M), f32),
        compiler_params=pltpu.CompilerParams(
            dimension_semantics=("parallel", "arbitrary"),
        ),
        interpret=interpret,
        name="mhca_wnorm",
    )(logits, srow)

    out = pl.pallas_call(
        _out_kernel,
        out_shape=jax.ShapeDtypeStruct((_B, _I), f32),
        interpret=interpret,
        name="mhca_out",
    )(acc, srow, st, Wo, bo.reshape(1, _I))

    return out, w
